# baseline (speedup 1.0000x reference)
import jax
import jax.numpy as jnp
from jax import lax
from jax.experimental import pallas as pl
from jax.experimental.pallas import tpu as pltpu
from jax.experimental.pallas import tpu_sc as plsc

BATCH = 16384
VOCAB = 1000000
DIM = 64

_NC = 1
_NS = 16
_NW = _NC * _NS
_BPW = BATCH // _NW
_CHUNK = 128
_NCH = _BPW // _CHUNK


_HALF = _BPW // 2
_NCHH = _HALF // _CHUNK


def _body(feat_a_hbm, feat_b_hbm, wa_hbm, wb_hbm, out_hbm,
          idx_v, a_v, b_v, sem):
    wid = lax.axis_index("s") * _NC + lax.axis_index("c")

    for h in range(2):
        base = wid * _BPW + h * _HALF
        for j in range(_NCHH):
            pltpu.sync_copy(feat_a_hbm.at[pl.ds(base + j * _CHUNK, _CHUNK)],
                            idx_v.at[0, j])
            pltpu.sync_copy(feat_b_hbm.at[pl.ds(base + j * _CHUNK, _CHUNK)],
                            idx_v.at[1, j])
        copies = []
        for j in range(_NCHH):
            copies.append(pltpu.async_copy(
                wa_hbm.at[idx_v.at[0, j]],
                a_v.at[pl.ds(j * _CHUNK, _CHUNK)], sem))
            copies.append(pltpu.async_copy(
                wb_hbm.at[idx_v.at[1, j]],
                b_v.at[pl.ds(j * _CHUNK, _CHUNK)], sem))
        for c in copies:
            c.wait()
        pltpu.sync_copy(a_v, out_hbm.at[pl.ds(base, _HALF), 0])
        pltpu.sync_copy(b_v, out_hbm.at[pl.ds(base, _HALF), 1])


@jax.jit
def kernel(feat_a, feat_b, W_a, W_b):
    mesh = plsc.VectorSubcoreMesh(core_axis_name="c", subcore_axis_name="s",
                                  num_cores=_NC)
    out = pl.kernel(
        _body,
        mesh=mesh,
        out_type=jax.ShapeDtypeStruct((BATCH, 2, DIM), jnp.float32),
        scratch_types=[
            pltpu.VMEM((2, _NCHH, _CHUNK), jnp.int32),
            pltpu.VMEM((_HALF, DIM), jnp.float32),
            pltpu.VMEM((_HALF, DIM), jnp.float32),
            pltpu.SemaphoreType.DMA,
        ],
        compiler_params=pltpu.CompilerParams(use_tc_tiling_on_sc=False),
    )(feat_a, feat_b, W_a, W_b)
    return out.reshape(BATCH, 2 * DIM)
